# baseline (speedup 1.0000x reference)
import functools

import jax
import jax.numpy as jnp
from jax import lax
from jax.experimental import pallas as pl
from jax.experimental.pallas import tpu as pltpu
from jax.experimental.pallas import tpu_sc as plsc

_F = 26
_V = 100000
_D = 16
_B = 4096
_T = 20

_NC = 2
_NS = 16
_NW = _NC * _NS
_BW = _B // _NW
_LPS = _BW * _T
_GC = 32
_NCH = _BW // _GC
_RPC = _GC * _T
_JPC = _RPC // 128


def _sc_body(idx2_hbm, val_hbm, emb_hbm, wide_hbm, oemb_hbm,
             owide_hbm, idx2_v, val_v, rows_v, w_v, oemb_v, owide_v,
             sem_e0, sem_w0, sem_e1, sem_w1):
    wid = lax.axis_index("s") * _NC + lax.axis_index("c")
    b0 = wid * _BW
    sems = ((sem_e0, sem_w0), (sem_e1, sem_w1))

    def step(f, carry):
        i0 = (f * _B + b0) * _T

        pltpu.sync_copy(idx2_hbm.at[pl.ds(i0, _LPS)], idx2_v)
        pltpu.sync_copy(val_hbm.at[pl.ds(i0, _LPS)], val_v)

        emb_f = emb_hbm.at[f]
        wide_f = wide_hbm.at[f]

        def issue(c):
            sem_e, sem_w = sems[c % 2]
            cps = []
            for j in range(_JPC):
                o = c * _RPC + j * 128
                cps.append(pltpu.async_copy(
                    emb_f.at[idx2_v.at[pl.ds(o, 128)]],
                    rows_v.at[pl.ds(o, 128)], sem_e))
                cps.append(pltpu.async_copy(
                    wide_f.at[idx2_v.at[pl.ds(o, 128)]],
                    w_v.at[pl.ds(o, 128)], sem_w))
            return cps

        inflight = [issue(0), None]
        for c in range(_NCH):
            if c + 1 < _NCH:
                inflight[(c + 1) % 2] = issue(c + 1)
            for cp in inflight[c % 2]:
                cp.wait()

            base = c * _RPC
            def grp(gl, c2, c=c, base=base):
                g = c * _GC + gl
                b = g * _T
                lo = val_v[pl.ds(b, 16)]
                hi = val_v[pl.ds(b + 4, 16)]
                acc = rows_v[base + gl, :] * lo[0]
                for t in range(1, 16):
                    acc = acc + rows_v[base + t * _GC + gl, :] * lo[t]
                for t in range(16, _T):
                    acc = acc + rows_v[base + t * _GC + gl, :] * hi[t - 4]
                oemb_v[g, :] = acc
                return c2

            lax.fori_loop(0, _GC, grp, 0)

            for oo in range(0, _GC, 16):
                acc = w_v[pl.ds(base + oo, 16)]
                for t in range(1, _T):
                    acc = acc + w_v[pl.ds(base + t * _GC + oo, 16)]
                owide_v[pl.ds(c * _GC + oo, 16)] = acc

        pltpu.sync_copy(oemb_v, oemb_hbm.at[pl.ds(f * _B + b0, _BW), :])
        pltpu.sync_copy(owide_v, owide_hbm.at[pl.ds(f * _B + b0, _BW)])
        return carry

    lax.fori_loop(0, _F, step, 0)


_sc_pool = functools.partial(
    pl.kernel,
    out_type=[jax.ShapeDtypeStruct((_F * _B, _D), jnp.float32),
              jax.ShapeDtypeStruct((_F * _B,), jnp.float32)],
    mesh=plsc.VectorSubcoreMesh(core_axis_name="c", subcore_axis_name="s"),
    compiler_params=pltpu.CompilerParams(use_tc_tiling_on_sc=False),
    scratch_types=[
        pltpu.VMEM((_LPS,), jnp.int32),
        pltpu.VMEM((_LPS,), jnp.float32),
        pltpu.VMEM((_LPS, _D), jnp.float32),
        pltpu.VMEM((_LPS,), jnp.float32),
        pltpu.VMEM((_BW, _D), jnp.float32),
        pltpu.VMEM((_BW,), jnp.float32),
        pltpu.SemaphoreType.DMA,
        pltpu.SemaphoreType.DMA,
        pltpu.SemaphoreType.DMA,
        pltpu.SemaphoreType.DMA,
    ],
)(_sc_body)


def kernel(feat_0_index, feat_0_value, feat_1_index, feat_1_value, feat_2_index, feat_2_value, feat_3_index, feat_3_value, feat_4_index, feat_4_value, feat_5_index, feat_5_value, feat_6_index, feat_6_value, feat_7_index, feat_7_value, feat_8_index, feat_8_value, feat_9_index, feat_9_value, feat_10_index, feat_10_value, feat_11_index, feat_11_value, feat_12_index, feat_12_value, feat_13_index, feat_13_value, feat_14_index, feat_14_value, feat_15_index, feat_15_value, feat_16_index, feat_16_value, feat_17_index, feat_17_value, feat_18_index, feat_18_value, feat_19_index, feat_19_value, feat_20_index, feat_20_value, feat_21_index, feat_21_value, feat_22_index, feat_22_value, feat_23_index, feat_23_value, feat_24_index, feat_24_value, feat_25_index, feat_25_value, emb_tables, wide_tables):
    feats = list(locals().values())
    idxs = [feats[2 * i] for i in range(_F)]
    vals = [feats[2 * i + 1] for i in range(_F)]

    idx = jnp.stack([a.reshape(_B * _T) for a in idxs]).reshape(-1)
    val = jnp.stack([a.reshape(_B * _T) for a in vals]).reshape(-1)
    idx2 = (idx.reshape(_F, _B // _BW, _NCH, _GC, _T)
            .transpose(0, 1, 2, 4, 3).reshape(-1))

    oemb, owide = _sc_pool(idx2, val, emb_tables, wide_tables)
    emb_tensor = oemb.reshape(_F, _B, _D).transpose(1, 0, 2)
    wide_tensor = owide.reshape(_F, _B).T
    return (wide_tensor, emb_tensor)


# speedup vs baseline: 1.0341x; 1.0341x over previous
import functools

import jax
import jax.numpy as jnp
from jax import lax
from jax.experimental import pallas as pl
from jax.experimental.pallas import tpu as pltpu
from jax.experimental.pallas import tpu_sc as plsc

_F = 26
_V = 100000
_D = 16
_B = 4096
_T = 20

_NC = 2
_NS = 16
_NW = _NC * _NS
_BW = _B // _NW
_LPS = _BW * _T
_NJ = _LPS // 128


def _sc_body(idx2_hbm, val_hbm, emb_hbm, wide_hbm, oemb_hbm,
             owide_hbm, idx2_v, val_v, rows_v, w_v, oemb_v, owide_v,
             sem_e, sem_w):
    wid = lax.axis_index("s") * _NC + lax.axis_index("c")
    b0 = wid * _BW

    def step(f, carry):
        i0 = (f * _B + b0) * _T

        pltpu.sync_copy(idx2_hbm.at[pl.ds(i0, _LPS)], idx2_v)
        pltpu.sync_copy(val_hbm.at[pl.ds(i0, _LPS)], val_v)

        emb_f = emb_hbm.at[f]
        wide_f = wide_hbm.at[f]
        cps = []
        for j in range(_NJ):
            cps.append(pltpu.async_copy(
                emb_f.at[idx2_v.at[pl.ds(j * 128, 128)]],
                rows_v.at[pl.ds(j * 128, 128)], sem_e))
            cps.append(pltpu.async_copy(
                wide_f.at[idx2_v.at[pl.ds(j * 128, 128)]],
                w_v.at[pl.ds(j * 128, 128)], sem_w))
        for c in cps:
            c.wait()

        def grp(g, c2):
            b = g * _T
            lo = val_v[pl.ds(b, 16)]
            hi = val_v[pl.ds(b + 4, 16)]
            acc = [rows_v[t * 128 + g, :] * lo[t] for t in range(4)]
            for t in range(4, 16):
                acc[t % 4] = acc[t % 4] + rows_v[t * 128 + g, :] * lo[t]
            for t in range(16, _T):
                acc[t % 4] = acc[t % 4] + rows_v[t * 128 + g, :] * hi[t - 4]
            oemb_v[g, :] = (acc[0] + acc[1]) + (acc[2] + acc[3])
            return c2

        lax.fori_loop(0, _BW, grp, 0)

        def wgrp(gb, c2):
            o = gb * 16
            acc = w_v[pl.ds(o, 16)]
            for t in range(1, _T):
                acc = acc + w_v[pl.ds(t * 128 + o, 16)]
            owide_v[pl.ds(o, 16)] = acc
            return c2

        lax.fori_loop(0, _BW // 16, wgrp, 0)

        pltpu.sync_copy(oemb_v, oemb_hbm.at[pl.ds(f * _B + b0, _BW), :])
        pltpu.sync_copy(owide_v, owide_hbm.at[pl.ds(f * _B + b0, _BW)])
        return carry

    lax.fori_loop(0, _F, step, 0)


_sc_pool = functools.partial(
    pl.kernel,
    out_type=[jax.ShapeDtypeStruct((_F * _B, _D), jnp.float32),
              jax.ShapeDtypeStruct((_F * _B,), jnp.float32)],
    mesh=plsc.VectorSubcoreMesh(core_axis_name="c", subcore_axis_name="s"),
    compiler_params=pltpu.CompilerParams(use_tc_tiling_on_sc=False),
    scratch_types=[
        pltpu.VMEM((_LPS,), jnp.int32),
        pltpu.VMEM((_LPS,), jnp.float32),
        pltpu.VMEM((_LPS, _D), jnp.float32),
        pltpu.VMEM((_LPS,), jnp.float32),
        pltpu.VMEM((_BW, _D), jnp.float32),
        pltpu.VMEM((_BW,), jnp.float32),
        pltpu.SemaphoreType.DMA,
        pltpu.SemaphoreType.DMA,
    ],
)(_sc_body)


def kernel(feat_0_index, feat_0_value, feat_1_index, feat_1_value, feat_2_index, feat_2_value, feat_3_index, feat_3_value, feat_4_index, feat_4_value, feat_5_index, feat_5_value, feat_6_index, feat_6_value, feat_7_index, feat_7_value, feat_8_index, feat_8_value, feat_9_index, feat_9_value, feat_10_index, feat_10_value, feat_11_index, feat_11_value, feat_12_index, feat_12_value, feat_13_index, feat_13_value, feat_14_index, feat_14_value, feat_15_index, feat_15_value, feat_16_index, feat_16_value, feat_17_index, feat_17_value, feat_18_index, feat_18_value, feat_19_index, feat_19_value, feat_20_index, feat_20_value, feat_21_index, feat_21_value, feat_22_index, feat_22_value, feat_23_index, feat_23_value, feat_24_index, feat_24_value, feat_25_index, feat_25_value, emb_tables, wide_tables):
    feats = list(locals().values())
    idxs = [feats[2 * i] for i in range(_F)]
    vals = [feats[2 * i + 1] for i in range(_F)]

    idx = jnp.stack([a.reshape(_B * _T) for a in idxs]).reshape(-1)
    val = jnp.stack([a.reshape(_B * _T) for a in vals]).reshape(-1)
    idx2 = (idx.reshape(_F, _B // _BW, _BW, _T)
            .transpose(0, 1, 3, 2).reshape(-1))

    oemb, owide = _sc_pool(idx2, val, emb_tables, wide_tables)
    emb_tensor = oemb.reshape(_F, _B, _D).transpose(1, 0, 2)
    wide_tensor = owide.reshape(_F, _B).T
    return (wide_tensor, emb_tensor)


# speedup vs baseline: 1.0427x; 1.0083x over previous
import functools

import jax
import jax.numpy as jnp
from jax import lax
from jax.experimental import pallas as pl
from jax.experimental.pallas import tpu as pltpu
from jax.experimental.pallas import tpu_sc as plsc

_F = 26
_V = 100000
_D = 16
_B = 4096
_T = 20

_NC = 2
_NS = 16
_NW = _NC * _NS
_BW = _B // _NW
_LPS = _BW * _T
_NJ = _LPS // 128


def _sc_body(idx2_hbm, val_hbm, emb_hbm, wide_hbm, oemb_hbm,
             owide_hbm, idx2_v, val_v, rows_v, w_v, oemb_v, owide_v,
             sem_e, sem_w):
    wid = lax.axis_index("s") * _NC + lax.axis_index("c")
    b0 = wid * _BW

    def step(f, carry):
        i0 = (f * _B + b0) * _T

        pltpu.sync_copy(idx2_hbm.at[pl.ds(i0, _LPS)], idx2_v)
        pltpu.sync_copy(val_hbm.at[pl.ds(i0, _LPS)], val_v)

        emb_f = emb_hbm.at[f]
        wide_f = wide_hbm.at[f]
        cps = []
        for j in range(_NJ):
            cps.append(pltpu.async_copy(
                emb_f.at[idx2_v.at[pl.ds(j * 128, 128)]],
                rows_v.at[pl.ds(j * 128, 128)], sem_e))
            cps.append(pltpu.async_copy(
                wide_f.at[idx2_v.at[pl.ds(j * 128, 128)]],
                w_v.at[pl.ds(j * 128, 128)], sem_w))
        for c in cps:
            c.wait()

        def grp(gi, c2):
            for k in range(2):
                g = gi * 2 + k
                b = g * _T
                lo = val_v[pl.ds(b, 16)]
                hi = val_v[pl.ds(b + 4, 16)]
                acc = [rows_v[t * 128 + g, :] * lo[t] for t in range(4)]
                for t in range(4, 16):
                    acc[t % 4] = acc[t % 4] + rows_v[t * 128 + g, :] * lo[t]
                for t in range(16, _T):
                    acc[t % 4] = acc[t % 4] + rows_v[t * 128 + g, :] * hi[t - 4]
                oemb_v[g, :] = (acc[0] + acc[1]) + (acc[2] + acc[3])
            return c2

        lax.fori_loop(0, _BW // 2, grp, 0)

        def wgrp(gb, c2):
            o = gb * 16
            acc = w_v[pl.ds(o, 16)]
            for t in range(1, _T):
                acc = acc + w_v[pl.ds(t * 128 + o, 16)]
            owide_v[pl.ds(o, 16)] = acc
            return c2

        lax.fori_loop(0, _BW // 16, wgrp, 0)

        pltpu.sync_copy(oemb_v, oemb_hbm.at[pl.ds(f * _B + b0, _BW), :])
        pltpu.sync_copy(owide_v, owide_hbm.at[pl.ds(f * _B + b0, _BW)])
        return carry

    lax.fori_loop(0, _F, step, 0)


_sc_pool = functools.partial(
    pl.kernel,
    out_type=[jax.ShapeDtypeStruct((_F * _B, _D), jnp.float32),
              jax.ShapeDtypeStruct((_F * _B,), jnp.float32)],
    mesh=plsc.VectorSubcoreMesh(core_axis_name="c", subcore_axis_name="s"),
    compiler_params=pltpu.CompilerParams(use_tc_tiling_on_sc=False),
    scratch_types=[
        pltpu.VMEM((_LPS,), jnp.int32),
        pltpu.VMEM((_LPS,), jnp.float32),
        pltpu.VMEM((_LPS, _D), jnp.float32),
        pltpu.VMEM((_LPS,), jnp.float32),
        pltpu.VMEM((_BW, _D), jnp.float32),
        pltpu.VMEM((_BW,), jnp.float32),
        pltpu.SemaphoreType.DMA,
        pltpu.SemaphoreType.DMA,
    ],
)(_sc_body)


def kernel(feat_0_index, feat_0_value, feat_1_index, feat_1_value, feat_2_index, feat_2_value, feat_3_index, feat_3_value, feat_4_index, feat_4_value, feat_5_index, feat_5_value, feat_6_index, feat_6_value, feat_7_index, feat_7_value, feat_8_index, feat_8_value, feat_9_index, feat_9_value, feat_10_index, feat_10_value, feat_11_index, feat_11_value, feat_12_index, feat_12_value, feat_13_index, feat_13_value, feat_14_index, feat_14_value, feat_15_index, feat_15_value, feat_16_index, feat_16_value, feat_17_index, feat_17_value, feat_18_index, feat_18_value, feat_19_index, feat_19_value, feat_20_index, feat_20_value, feat_21_index, feat_21_value, feat_22_index, feat_22_value, feat_23_index, feat_23_value, feat_24_index, feat_24_value, feat_25_index, feat_25_value, emb_tables, wide_tables):
    feats = list(locals().values())
    idxs = [feats[2 * i] for i in range(_F)]
    vals = [feats[2 * i + 1] for i in range(_F)]

    idx = jnp.stack([a.reshape(_B * _T) for a in idxs]).reshape(-1)
    val = jnp.stack([a.reshape(_B * _T) for a in vals]).reshape(-1)
    idx2 = (idx.reshape(_F, _B // _BW, _BW, _T)
            .transpose(0, 1, 3, 2).reshape(-1))

    oemb, owide = _sc_pool(idx2, val, emb_tables, wide_tables)
    emb_tensor = oemb.reshape(_F, _B, _D).transpose(1, 0, 2)
    wide_tensor = owide.reshape(_F, _B).T
    return (wide_tensor, emb_tensor)
